# baseline (device time: 46032 ns/iter reference)
import jax
import jax.numpy as jnp
from jax import lax
from jax.experimental import pallas as pl
from jax.experimental.pallas import tpu as pltpu

N_DEV = 8
M = 1536
N = 1536
SEG = M // N_DEV
NC = 6
CW = N // NC
SIZES = (768, 384, 192)
RBASE = (0, 768, 1152)


def _gelu(z):
    return 0.5 * z * (1.0 + jnp.tanh(0.7978845608 * (z + 0.044715 * z * z * z)))


def _make_schedule(dims):
    bases = [0]
    rs = []
    b = 0
    for i, (bit, partner) in enumerate(dims):
        ln = SIZES[i]
        rs.append((partner, ln, b + (1 - bit) * ln, b + bit * ln))
        b = b + bit * ln
        bases.append(b)
    ag = []
    for i in range(3):
        ln = SIZES[2 - i]
        my_base = bases[3 - i]
        parent = bases[2 - i]
        ag.append((dims[2 - i][1], ln, my_base, 2 * parent + ln - my_base))
    return rs, ag, bases[3]


def kernel(A, B):
    K = A.shape[1]

    def body(a_ref, b_ref, out_ref, a16, b16, rs_recv, send_sems, recv_sems):
        p = lax.axis_index("i")
        cz = (p >> 2) & 1
        cy = (p >> 1) & 1
        cx = (p & 1) ^ cy
        pz = p ^ 4
        px = (p & 4) | ((p & 3) ^ 1)
        py = (p & 4) | (3 - (p & 3))

        orders = [
            [(cz, pz), (cx, px), (cy, py)],
            [(cx, px), (cy, py), (cz, pz)],
            [(cy, py), (cz, pz), (cx, px)],
        ]
        scheds = [_make_schedule(orders[c % 3]) for c in range(NC)]

        def rs_rdma(c, s):
            partner, ln, send_base, _ = scheds[c][0][s]
            col = CW * c
            return pltpu.make_async_remote_copy(
                src_ref=out_ref.at[pl.ds(send_base, ln), pl.ds(col, CW)],
                dst_ref=rs_recv.at[pl.ds(RBASE[s], ln), pl.ds(col, CW)],
                send_sem=send_sems.at[s * NC + c],
                recv_sem=recv_sems.at[s * NC + c],
                device_id=(partner,),
                device_id_type=pl.DeviceIdType.MESH,
            )

        def ag_rdma(c, s):
            partner, ln, base, _ = scheds[c][1][s]
            col = CW * c
            return pltpu.make_async_remote_copy(
                src_ref=out_ref.at[pl.ds(base, ln), pl.ds(col, CW)],
                dst_ref=out_ref.at[pl.ds(base, ln), pl.ds(col, CW)],
                send_sem=send_sems.at[18 + s * NC + c],
                recv_sem=recv_sems.at[18 + s * NC + c],
                device_id=(partner,),
                device_id_type=pl.DeviceIdType.MESH,
            )

        barrier = pltpu.get_barrier_semaphore()
        for nbr in (pz, px, py):
            pl.semaphore_signal(
                barrier, inc=1, device_id=(nbr,),
                device_id_type=pl.DeviceIdType.MESH,
            )
        pl.semaphore_wait(barrier, 3)

        all_rdmas = []
        rs_infl = [None] * NC

        a16[:, :] = a_ref[:, :].astype(jnp.bfloat16)

        for c in range(NC):
            _, ln, send_base, keep_base = scheds[c][0][0]
            col = CW * c
            b16[:, pl.ds(col, CW)] = b_ref[:, pl.ds(col, CW)].astype(
                jnp.bfloat16
            )
            out_ref[pl.ds(send_base, ln), pl.ds(col, CW)] = jnp.dot(
                a16[pl.ds(send_base, ln), :], b16[:, pl.ds(col, CW)],
                preferred_element_type=jnp.float32,
            ).astype(jnp.bfloat16)
            rdma = rs_rdma(c, 0)
            rdma.start()
            rs_infl[c] = rdma
            all_rdmas.append(rdma)
            out_ref[pl.ds(keep_base, ln), pl.ds(col, CW)] = jnp.dot(
                a16[pl.ds(keep_base, ln), :], b16[:, pl.ds(col, CW)],
                preferred_element_type=jnp.float32,
            ).astype(jnp.bfloat16)

        for s in range(2):
            for c in range(NC):
                _, ln, _, keep_base = scheds[c][0][s]
                _, ln_n, send_base_n, keep_base_n = scheds[c][0][s + 1]
                col = CW * c
                rs_infl[c].wait_recv()
                r_send = RBASE[s] + (send_base_n - keep_base)
                r_keep = RBASE[s] + (keep_base_n - keep_base)
                out_ref[pl.ds(send_base_n, ln_n), pl.ds(col, CW)] = (
                    out_ref[pl.ds(send_base_n, ln_n), pl.ds(col, CW)]
                    + rs_recv[pl.ds(r_send, ln_n), pl.ds(col, CW)]
                )
                rdma = rs_rdma(c, s + 1)
                rdma.start()
                rs_infl[c] = rdma
                all_rdmas.append(rdma)
                out_ref[pl.ds(keep_base_n, ln_n), pl.ds(col, CW)] = (
                    out_ref[pl.ds(keep_base_n, ln_n), pl.ds(col, CW)]
                    + rs_recv[pl.ds(r_keep, ln_n), pl.ds(col, CW)]
                )

        ag_infl = [None] * NC
        for c in range(NC):
            seg = scheds[c][2]
            col = CW * c
            rs_infl[c].wait_recv()
            zv = (
                out_ref[pl.ds(seg, SEG), pl.ds(col, CW)].astype(jnp.float32)
                + rs_recv[pl.ds(RBASE[2], SEG), pl.ds(col, CW)].astype(
                    jnp.float32
                )
            )
            out_ref[pl.ds(seg, SEG), pl.ds(col, CW)] = _gelu(zv).astype(
                jnp.bfloat16
            )
            rdma = ag_rdma(c, 0)
            rdma.start()
            ag_infl[c] = rdma
            all_rdmas.append(rdma)

        for s in range(3):
            for c in range(NC):
                ag_infl[c].wait_recv()
                if s < 2:
                    rdma = ag_rdma(c, s + 1)
                    rdma.start()
                    ag_infl[c] = rdma
                    all_rdmas.append(rdma)

        for rdma in all_rdmas:
            rdma.wait_send()

    return pl.pallas_call(
        body,
        out_shape=jax.ShapeDtypeStruct((M, N), jnp.bfloat16),
        in_specs=[
            pl.BlockSpec(memory_space=pltpu.VMEM),
            pl.BlockSpec(memory_space=pltpu.VMEM),
        ],
        out_specs=pl.BlockSpec(memory_space=pltpu.VMEM),
        scratch_shapes=[
            pltpu.VMEM((M, K), jnp.bfloat16),
            pltpu.VMEM((K, N), jnp.bfloat16),
            pltpu.VMEM((1344, N), jnp.bfloat16),
            pltpu.SemaphoreType.DMA((36,)),
            pltpu.SemaphoreType.DMA((36,)),
        ],
        compiler_params=pltpu.CompilerParams(collective_id=5),
    )(A, B)
